# baseline (speedup 1.0000x reference)
import math

import jax
import jax.numpy as jnp
from jax import lax
from jax.experimental import pallas as pl
from jax.experimental.pallas import tpu as pltpu
from jax.experimental.pallas import tpu_sc as plsc

D = 32
L = 16
NC, NS = 2, 16
NW = NC * NS
S = 128
SCALE = math.sqrt(float(D))

_TBV = 2048


def _transpose_block(wt_ref, out_ref):
    xt = wt_ref[...].T
    out_ref[...] = jnp.concatenate(
        [xt[512 * k:512 * (k + 1)] for k in range(4)], axis=1
    )


def _repack_weight(weight):
    V = weight.shape[0]
    wt = weight.T
    nblk = (V + _TBV - 1) // _TBV
    w128 = pl.pallas_call(
        _transpose_block,
        grid=(nblk,),
        in_specs=[pl.BlockSpec((D, _TBV), lambda i: (0, i))],
        out_specs=pl.BlockSpec((512, 128), lambda i: (i, 0)),
        out_shape=jax.ShapeDtypeStruct((nblk * 512, 128), jnp.float32),
    )(wt)
    return w128.reshape(nblk * 2048, 32)


def _emb_kernel(NT: int, NB: int):
    assert NB == S * NW and NT % 2 == 0
    TSL = D * S

    mesh = plsc.VectorSubcoreMesh(core_axis_name="c", subcore_axis_name="s")

    @pl.kernel(
        out_type=jax.ShapeDtypeStruct((NT * NB * D,), jnp.float32),
        mesh=mesh,
        compiler_params=pltpu.CompilerParams(
            use_tc_tiling_on_sc=False, needs_layout_passes=False
        ),
        scratch_types=[
            pltpu.VMEM((NT, S), jnp.int32),
            pltpu.VMEM((S, D), jnp.float32),
            pltpu.VMEM((S, D), jnp.float32),
            pltpu.VMEM((TSL,), jnp.float32),
            pltpu.VMEM((TSL,), jnp.float32),
            pltpu.SemaphoreType.DMA,
            pltpu.SemaphoreType.DMA,
            pltpu.SemaphoreType.DMA,
            pltpu.SemaphoreType.DMA,
        ],
    )
    def body(tok_hbm, w_hbm, out_hbm, idx_v, r0, r1, t0, t1, g0, g1, o0, o1):
        wid = lax.axis_index("s") * NC + lax.axis_index("c")
        pltpu.sync_copy(tok_hbm.at[:, pl.ds(wid * S, S)], idx_v)

        @pl.loop(0, NT)
        def _remap_t(t):
            @pl.loop(0, S // L, unroll=8)
            def _remap(i):
                sl = pl.ds(i * L, L)
                v = idx_v[t, sl]
                idx_v[t, sl] = ((v >> 11) << 11) + ((v & 511) << 2) + ((v >> 9) & 3)

        rows = (r0, r1)
        tbuf = (t0, t1)
        gsem = (g0, g1)
        osem = (o0, o1)
        lane = lax.iota(jnp.int32, L)

        def fire(t, p):
            pltpu.async_copy(w_hbm.at[idx_v.at[t]], rows[p], gsem[p])

        def drain(t, p):
            pltpu.make_async_copy(
                w_hbm.at[idx_v.at[t]], rows[p], gsem[p]
            ).wait()

        def trans_scale(p):
            @pl.loop(0, D, unroll=4)
            def _col(d):
                col = jnp.full((L,), 0, jnp.int32) + d
                for j in range(S // L):
                    v = plsc.load_gather(rows[p], [lane + j * L, col])
                    tbuf[p][pl.ds(d * S + j * L, L)] = v * SCALE

        def out_start(t, p):
            for kd in range(D // 8):
                pltpu.async_copy(
                    tbuf[p].at[pl.ds(kd * 1024, 1024)],
                    out_hbm.at[pl.ds(t * (D * NB) + (kd * NW + wid) * 1024, 1024)],
                    osem[p],
                )

        def out_wait(t, p):
            for kd in range(D // 8):
                pltpu.make_async_copy(
                    tbuf[p].at[pl.ds(kd * 1024, 1024)],
                    out_hbm.at[pl.ds(t * (D * NB) + (kd * NW + wid) * 1024, 1024)],
                    osem[p],
                ).wait()

        fire(0, 0)
        fire(1, 1)
        drain(0, 0)
        trans_scale(0)
        out_start(0, 0)

        @pl.loop(0, (NT - 2) // 2)
        def _pipe(i):
            for b2 in range(2):
                t = 1 + 2 * i + b2
                p = (1 + b2) % 2
                q = 1 - p
                out_wait(t - 1, q)
                fire(t + 1, q)
                drain(t, p)
                trans_scale(p)
                out_start(t, p)

        out_wait(NT - 2, 0)
        drain(NT - 1, 1)
        trans_scale(1)
        out_start(NT - 1, 1)
        out_wait(NT - 1, 1)

    return body


def kernel(tokens, weight):
    n0, n1 = tokens.shape
    tok_t = tokens.T.astype(jnp.int32)
    o = _emb_kernel(n1, n0)(tok_t, _repack_weight(weight))
    view = o.reshape(n1, D // 8, n0 // 128, 8, 128)
    return view.transpose(2, 4, 0, 1, 3).reshape(n0, n1, D)


# speedup vs baseline: 1.0037x; 1.0037x over previous
import math

import jax
import jax.numpy as jnp
from jax import lax
from jax.experimental import pallas as pl
from jax.experimental.pallas import tpu as pltpu
from jax.experimental.pallas import tpu_sc as plsc

D = 32
L = 16
NC, NS = 2, 16
NW = NC * NS
S = 128
SCALE = math.sqrt(float(D))

_TBV = 2048


def _transpose_block(wt_ref, out_ref):
    xt = wt_ref[...].T
    out_ref[...] = jnp.concatenate(
        [xt[512 * k:512 * (k + 1)] for k in range(4)], axis=1
    )


def _repack_weight(weight):
    V = weight.shape[0]
    wt = weight.T
    nblk = (V + _TBV - 1) // _TBV
    w128 = pl.pallas_call(
        _transpose_block,
        grid=(nblk,),
        in_specs=[pl.BlockSpec((D, _TBV), lambda i: (0, i))],
        out_specs=pl.BlockSpec((512, 128), lambda i: (i, 0)),
        out_shape=jax.ShapeDtypeStruct((nblk * 512, 128), jnp.float32),
    )(wt)
    return w128.reshape(nblk * 2048, 32)


def _emb_kernel(NT: int, NB: int):
    NTG_W, NB_W = 4, 8
    NTG = NT // NTG_W
    BW = NB // NB_W
    NSTR = BW // S
    NKD = D // 8
    assert NTG * NTG_W == NT and BW * NB_W == NB
    assert NTG % 2 == 0 and NTG >= 4

    mesh = plsc.VectorSubcoreMesh(core_axis_name="c", subcore_axis_name="s")

    @pl.kernel(
        out_type=jax.ShapeDtypeStruct((NT * NB * D,), jnp.float32),
        mesh=mesh,
        compiler_params=pltpu.CompilerParams(
            use_tc_tiling_on_sc=False, needs_layout_passes=False
        ),
        scratch_types=[
            pltpu.VMEM((NTG, BW), jnp.int32),
            pltpu.VMEM((BW, D), jnp.float32),
            pltpu.VMEM((BW, D), jnp.float32),
            pltpu.VMEM((D * BW,), jnp.float32),
            pltpu.VMEM((D * BW,), jnp.float32),
            pltpu.SemaphoreType.DMA,
            pltpu.SemaphoreType.DMA,
            pltpu.SemaphoreType.DMA,
            pltpu.SemaphoreType.DMA,
        ],
    )
    def body(tok_hbm, w_hbm, out_hbm, idx_v, r0, r1, t0, t1, g0, g1, o0, o1):
        wid = lax.axis_index("s") * NC + lax.axis_index("c")
        wt = wid // NB_W
        wb = wid % NB_W
        pltpu.sync_copy(
            tok_hbm.at[pl.ds(wt * NTG, NTG), pl.ds(wb * BW, BW)], idx_v
        )

        @pl.loop(0, NTG)
        def _remap_t(t):
            @pl.loop(0, BW // L, unroll=8)
            def _remap(i):
                sl = pl.ds(i * L, L)
                v = idx_v[t, sl]
                idx_v[t, sl] = ((v >> 11) << 11) + ((v & 511) << 2) + ((v >> 9) & 3)

        rows = (r0, r1)
        tbuf = (t0, t1)
        gsem = (g0, g1)
        osem = (o0, o1)
        lane = lax.iota(jnp.int32, L)

        def fire(t, p):
            for k in range(NSTR):
                pltpu.async_copy(
                    w_hbm.at[idx_v.at[t, pl.ds(k * S, S)]],
                    rows[p].at[pl.ds(k * S, S)],
                    gsem[p],
                )

        def drain(t, p):
            for k in range(NSTR):
                pltpu.make_async_copy(
                    w_hbm.at[idx_v.at[t, pl.ds(k * S, S)]],
                    rows[p].at[pl.ds(k * S, S)],
                    gsem[p],
                ).wait()

        def trans_scale(p):
            @pl.loop(0, BW // L, unroll=2)
            def _grp(j):
                row_ids = lane + j * L
                dyn = (j >> 3) * 1024 + (j & 7) * L
                for d in range(D):
                    kd, ds = d // 8, d % 8
                    col = jnp.full((L,), d, jnp.int32)
                    v = plsc.load_gather(rows[p], [row_ids, col])
                    tbuf[p][pl.ds(kd * (8 * BW) + ds * 128 + dyn, L)] = v * SCALE

        def out_start(t, p):
            tg = wt * NTG + t
            for kd in range(NKD):
                pltpu.async_copy(
                    tbuf[p].at[pl.ds(kd * (8 * BW), 8 * BW)],
                    out_hbm.at[
                        pl.ds(tg * (D * NB) + kd * (8 * NB) + wb * (8 * BW), 8 * BW)
                    ],
                    osem[p],
                )

        def out_wait(t, p):
            tg = wt * NTG + t
            for kd in range(NKD):
                pltpu.make_async_copy(
                    tbuf[p].at[pl.ds(kd * (8 * BW), 8 * BW)],
                    out_hbm.at[
                        pl.ds(tg * (D * NB) + kd * (8 * NB) + wb * (8 * BW), 8 * BW)
                    ],
                    osem[p],
                ).wait()

        fire(0, 0)
        fire(1, 1)
        drain(0, 0)
        trans_scale(0)
        out_start(0, 0)

        @pl.loop(0, (NTG - 2) // 2)
        def _pipe(i):
            for b2 in range(2):
                t = 1 + 2 * i + b2
                p = (1 + b2) % 2
                q = 1 - p
                out_wait(t - 1, q)
                fire(t + 1, q)
                drain(t, p)
                trans_scale(p)
                out_start(t, p)

        out_wait(NTG - 2, 0)
        drain(NTG - 1, 1)
        trans_scale(1)
        out_start(NTG - 1, 1)
        out_wait(NTG - 1, 1)

    return body


def kernel(tokens, weight):
    n0, n1 = tokens.shape
    tok_t = tokens.T.astype(jnp.int32)
    o = _emb_kernel(n1, n0)(tok_t, _repack_weight(weight))
    view = o.reshape(n1, D // 8, n0 // 128, 8, 128)
    return view.transpose(2, 4, 0, 1, 3).reshape(n0, n1, D)


# speedup vs baseline: 1.3169x; 1.3121x over previous
import math

import jax
import jax.numpy as jnp
from jax import lax
from jax.experimental import pallas as pl
from jax.experimental.pallas import tpu as pltpu
from jax.experimental.pallas import tpu_sc as plsc

D = 32
L = 16
NC, NS = 2, 16
NW = NC * NS
S = 128
SCALE = math.sqrt(float(D))

_TBV = 2048


def _transpose_block(wt_ref, out_ref):
    xt = wt_ref[...].T
    out_ref[...] = jnp.concatenate(
        [xt[512 * k:512 * (k + 1)] for k in range(4)], axis=1
    )


def _repack_weight(weight):
    V = weight.shape[0]
    wt = weight.T
    nblk = (V + _TBV - 1) // _TBV
    w128 = pl.pallas_call(
        _transpose_block,
        grid=(nblk,),
        in_specs=[pl.BlockSpec((D, _TBV), lambda i: (0, i))],
        out_specs=pl.BlockSpec((512, 128), lambda i: (i, 0)),
        out_shape=jax.ShapeDtypeStruct((nblk * 512, 128), jnp.float32),
    )(wt)
    return w128.reshape(nblk * 2048, 32)


def _emb_kernel(NT: int, NB: int):
    NTG_W, NB_W = 4, 8
    NTG = NT // NTG_W
    BW = NB // NB_W
    NSTR = BW // S
    NKD = D // 8
    assert NTG * NTG_W == NT and BW * NB_W == NB
    assert NTG % 2 == 0 and NTG >= 4

    mesh = plsc.VectorSubcoreMesh(core_axis_name="c", subcore_axis_name="s")

    @pl.kernel(
        out_type=jax.ShapeDtypeStruct((NT * NB * D,), jnp.float32),
        mesh=mesh,
        compiler_params=pltpu.CompilerParams(
            use_tc_tiling_on_sc=False, needs_layout_passes=False
        ),
        scratch_types=[
            pltpu.VMEM((NTG, BW), jnp.int32),
            pltpu.VMEM((BW, D), jnp.float32),
            pltpu.VMEM((BW, D), jnp.float32),
            pltpu.VMEM((D * BW,), jnp.float32),
            pltpu.VMEM((D * BW,), jnp.float32),
            pltpu.SemaphoreType.DMA,
            pltpu.SemaphoreType.DMA,
            pltpu.SemaphoreType.DMA,
            pltpu.SemaphoreType.DMA,
        ],
    )
    def body(tok_hbm, w_hbm, out_hbm, idx_v, r0, r1, t0, t1, g0, g1, o0, o1):
        wid = lax.axis_index("s") * NC + lax.axis_index("c")
        wt = wid // NB_W
        wb = wid % NB_W
        pltpu.sync_copy(
            tok_hbm.at[pl.ds(wt * NTG, NTG), pl.ds(wb * BW, BW)], idx_v
        )

        @pl.loop(0, NTG)
        def _remap_t(t):
            @plsc.parallel_loop(0, BW // L, unroll=8)
            def _remap(i):
                sl = pl.ds(i * L, L)
                v = idx_v[t, sl]
                idx_v[t, sl] = ((v >> 11) << 11) + ((v & 511) << 2) + ((v >> 9) & 3)

        rows = (r0, r1)
        tbuf = (t0, t1)
        gsem = (g0, g1)
        osem = (o0, o1)
        lane = lax.iota(jnp.int32, L)

        def fire(t, p):
            for k in range(NSTR):
                pltpu.async_copy(
                    w_hbm.at[idx_v.at[t, pl.ds(k * S, S)]],
                    rows[p].at[pl.ds(k * S, S)],
                    gsem[p],
                )

        def drain(t, p):
            for k in range(NSTR):
                pltpu.make_async_copy(
                    w_hbm.at[idx_v.at[t, pl.ds(k * S, S)]],
                    rows[p].at[pl.ds(k * S, S)],
                    gsem[p],
                ).wait()

        def trans_scale(p):
            @plsc.parallel_loop(0, BW // L, unroll=2)
            def _grp(j):
                row_ids = lane + j * L
                dyn = (j >> 3) * 1024 + (j & 7) * L
                for d in range(D):
                    kd, ds = d // 8, d % 8
                    col = jnp.full((L,), d, jnp.int32)
                    v = plsc.load_gather(rows[p], [row_ids, col])
                    tbuf[p][pl.ds(kd * (8 * BW) + ds * 128 + dyn, L)] = v * SCALE

        def out_start(t, p):
            tg = wt * NTG + t
            for kd in range(NKD):
                pltpu.async_copy(
                    tbuf[p].at[pl.ds(kd * (8 * BW), 8 * BW)],
                    out_hbm.at[
                        pl.ds(tg * (D * NB) + kd * (8 * NB) + wb * (8 * BW), 8 * BW)
                    ],
                    osem[p],
                )

        def out_wait(t, p):
            tg = wt * NTG + t
            for kd in range(NKD):
                pltpu.make_async_copy(
                    tbuf[p].at[pl.ds(kd * (8 * BW), 8 * BW)],
                    out_hbm.at[
                        pl.ds(tg * (D * NB) + kd * (8 * NB) + wb * (8 * BW), 8 * BW)
                    ],
                    osem[p],
                ).wait()

        fire(0, 0)
        fire(1, 1)
        drain(0, 0)
        trans_scale(0)
        out_start(0, 0)

        @pl.loop(0, (NTG - 2) // 2)
        def _pipe(i):
            for b2 in range(2):
                t = 1 + 2 * i + b2
                p = (1 + b2) % 2
                q = 1 - p
                out_wait(t - 1, q)
                fire(t + 1, q)
                drain(t, p)
                trans_scale(p)
                out_start(t, p)

        out_wait(NTG - 2, 0)
        drain(NTG - 1, 1)
        trans_scale(1)
        out_start(NTG - 1, 1)
        out_wait(NTG - 1, 1)

    return body


def kernel(tokens, weight):
    n0, n1 = tokens.shape
    tok_t = tokens.T.astype(jnp.int32)
    o = _emb_kernel(n1, n0)(tok_t, _repack_weight(weight))
    view = o.reshape(n1, D // 8, n0 // 128, 8, 128)
    return view.transpose(2, 4, 0, 1, 3).reshape(n0, n1, D)
